# Initial kernel scaffold; baseline (speedup 1.0000x reference)
import jax
import jax.numpy as jnp
from jax.experimental import pallas as pl


def kernel(x, edge_index, batch, W0, b0, W1, b1, W2, b2, Wout, bout):
    raise NotImplementedError("write your pallas kernel here")


# speedup vs baseline: 10.8174x; 10.8174x over previous
import functools
import jax
import jax.numpy as jnp
from jax import lax
from jax.experimental import pallas as pl
from jax.experimental.pallas import tpu as pltpu
from jax.experimental.pallas import tpu_sc as plsc

N = 10000
E = 320000
D = 128
G = 64
OUT = 10

NC = 2
NS = 16
NW = NC * NS
CHUNK = 128
NCHUNKS = E // CHUNK
EPT = E // NW
RPT = 624
RPT_LAST = N - RPT * (NS - 1)

BLK = 1000
NB = N // BLK

_SC_MESH = plsc.VectorSubcoreMesh(core_axis_name="c", subcore_axis_name="s")


DEGW = 128


@functools.partial(
    pl.kernel,
    out_type=jax.ShapeDtypeStruct((NC, N, DEGW), jnp.float32),
    mesh=_SC_MESH,
    scratch_types=[
        pltpu.VMEM((CHUNK,), jnp.int32),
        pltpu.VMEM((CHUNK, DEGW), jnp.float32),
        pltpu.VMEM_SHARED((N, DEGW), jnp.float32),
    ],
)
def _sc_degree(dst_hbm, ones_hbm, znodes_hbm, degp_hbm, dbuf, ones_v, deg_t):
    cid = lax.axis_index("c")
    sid = lax.axis_index("s")
    wid = sid * NC + cid

    pltpu.sync_copy(ones_hbm, ones_v)

    @pl.when(sid == 0)
    def _zero():
        pltpu.sync_copy(znodes_hbm, deg_t)

    plsc.subcore_barrier()

    nj = (NCHUNKS + NW - 1 - wid) // NW

    def body(j, carry):
        base = (wid + j * NW) * CHUNK
        pltpu.sync_copy(dst_hbm.at[pl.ds(base, CHUNK)], dbuf)
        pltpu.sync_copy(ones_v, deg_t.at[dbuf], add=True)
        return carry

    lax.fori_loop(0, nj, body, 0)

    plsc.subcore_barrier()

    @pl.when(sid == 0)
    def _writeout():
        pltpu.sync_copy(deg_t, degp_hbm.at[cid])


@functools.partial(
    pl.kernel,
    out_type=jax.ShapeDtypeStruct((NC, N, D), jnp.float32),
    mesh=_SC_MESH,
    scratch_types=[
        pltpu.VMEM((CHUNK,), jnp.int32),
        pltpu.VMEM((CHUNK,), jnp.int32),
        pltpu.VMEM((CHUNK, D), jnp.float32),
        pltpu.VMEM_SHARED((N, D), jnp.float32),
        pltpu.SemaphoreType.DMA,
    ],
)
def _sc_message(hws_hbm, src_hbm, dst_hbm, zrows_hbm, accp_hbm,
                sbuf, dbuf, rows, acc, sem):
    cid = lax.axis_index("c")
    sid = lax.axis_index("s")
    wid = sid * NC + cid

    @pl.when(sid < NS - 1)
    def _zero():
        pltpu.sync_copy(zrows_hbm.at[pl.ds(0, RPT), :],
                        acc.at[pl.ds(sid * RPT, RPT), :])

    @pl.when(sid == NS - 1)
    def _zero_last():
        pltpu.sync_copy(zrows_hbm, acc.at[pl.ds(RPT * (NS - 1), RPT_LAST), :])

    plsc.subcore_barrier()

    nj = (NCHUNKS + NW - 1 - wid) // NW

    def body(j, carry):
        base = (wid + j * NW) * CHUNK
        pltpu.sync_copy(src_hbm.at[pl.ds(base, CHUNK)], sbuf)
        pltpu.sync_copy(dst_hbm.at[pl.ds(base, CHUNK)], dbuf)
        pltpu.async_copy(hws_hbm.at[sbuf], rows, sem).wait()
        pltpu.sync_copy(rows, acc.at[dbuf], add=True)
        return carry

    lax.fori_loop(0, nj, body, 0)

    plsc.subcore_barrier()

    @pl.when(sid < NS - 1)
    def _writeout():
        pltpu.sync_copy(acc.at[pl.ds(sid * RPT, RPT), :],
                        accp_hbm.at[cid, pl.ds(sid * RPT, RPT), :])

    @pl.when(sid == NS - 1)
    def _writeout_last():
        base = RPT * (NS - 1)
        pltpu.sync_copy(acc.at[pl.ds(base, RPT_LAST), :],
                        accp_hbm.at[cid, pl.ds(base, RPT_LAST), :])


def _t0_body(deg_ref, x_ref, w_ref, dis_ref, hws_ref):
    d = jnp.sum(deg_ref[...], axis=1, keepdims=True) + 1.0
    dis = lax.rsqrt(d)
    dis_ref[...] = dis
    hws_ref[...] = dis * jnp.dot(x_ref[...], w_ref[...],
                                 preferred_element_type=jnp.float32, precision=lax.Precision.HIGHEST)


def _tc_first(degT, x, W0):
    return pl.pallas_call(
        _t0_body,
        grid=(NB,),
        in_specs=[
            pl.BlockSpec((BLK, NC), lambda i: (i, 0)),
            pl.BlockSpec((BLK, D), lambda i: (i, 0)),
            pl.BlockSpec((D, D), lambda i: (0, 0)),
        ],
        out_specs=[
            pl.BlockSpec((BLK, 1), lambda i: (i, 0)),
            pl.BlockSpec((BLK, D), lambda i: (i, 0)),
        ],
        out_shape=[
            jax.ShapeDtypeStruct((N, 1), jnp.float32),
            jax.ShapeDtypeStruct((N, D), jnp.float32),
        ],
    )(degT, x, W0)


def _tl_body(accp_ref, hws_ref, dis_ref, b_ref, w_ref, out_ref):
    acc = accp_ref[0] + accp_ref[1]
    dis = dis_ref[...]
    h = jnp.maximum(dis * (acc + hws_ref[...]) + b_ref[...], 0.0)
    out_ref[...] = dis * jnp.dot(h, w_ref[...],
                                 preferred_element_type=jnp.float32, precision=lax.Precision.HIGHEST)


def _tc_layer(accp, hws, dis, b, Wn):
    return pl.pallas_call(
        _tl_body,
        grid=(NB,),
        in_specs=[
            pl.BlockSpec((NC, BLK, D), lambda i: (0, i, 0)),
            pl.BlockSpec((BLK, D), lambda i: (i, 0)),
            pl.BlockSpec((BLK, 1), lambda i: (i, 0)),
            pl.BlockSpec((1, D), lambda i: (0, 0)),
            pl.BlockSpec((D, D), lambda i: (0, 0)),
        ],
        out_specs=pl.BlockSpec((BLK, D), lambda i: (i, 0)),
        out_shape=jax.ShapeDtypeStruct((N, D), jnp.float32),
    )(accp, hws, dis, b, Wn)


def _t3_body(accp_ref, hws_ref, dis_ref, b_ref, br_ref, w_ref, bo_ref,
             out_ref, gmax, gsum, cnt):
    step = pl.program_id(0)

    @pl.when(step == 0)
    def _init():
        gmax[...] = jnp.full((G, D), -jnp.inf, jnp.float32)
        gsum[...] = jnp.zeros((G, D), jnp.float32)
        cnt[...] = jnp.zeros((G, D), jnp.float32)

    acc = accp_ref[0] + accp_ref[1]
    dis = dis_ref[...]
    h = jnp.maximum(dis * (acc + hws_ref[...]) + b_ref[...], 0.0)

    br = br_ref[...]
    onehot = (br == lax.broadcasted_iota(jnp.int32, (BLK, G), 1)
              ).astype(jnp.float32)
    dgen = (((0,), (0,)), ((), ()))
    gsum[...] += lax.dot_general(onehot, h, dgen,
                                 preferred_element_type=jnp.float32, precision=lax.Precision.HIGHEST)
    cnt[...] += lax.dot_general(onehot, jnp.ones((BLK, D), jnp.float32),
                                dgen, preferred_element_type=jnp.float32, precision=lax.Precision.HIGHEST)

    def gbody(g, carry):
        m = br == g
        v = jnp.max(jnp.where(m, h, -jnp.inf), axis=0, keepdims=True)
        gmax[pl.ds(g, 1), :] = jnp.maximum(gmax[pl.ds(g, 1), :], v)
        return carry

    lax.fori_loop(0, G, gbody, 0)

    @pl.when(step == NB - 1)
    def _head():
        c = cnt[...]
        gmaxf = jnp.where(c > 0, gmax[...], 0.0)
        gs = gsum[...]
        gmean = gs / jnp.maximum(c, 1.0)
        w = w_ref[...]
        out_ref[...] = (
            jnp.dot(gmaxf, w[0:D, :], preferred_element_type=jnp.float32, precision=lax.Precision.HIGHEST)
            + jnp.dot(gmean, w[D:2 * D, :], preferred_element_type=jnp.float32, precision=lax.Precision.HIGHEST)
            + jnp.dot(gs, w[2 * D:3 * D, :], preferred_element_type=jnp.float32, precision=lax.Precision.HIGHEST)
            + bo_ref[...])


def _tc_pool_head(accp, hws, dis, b, br, Wp, bop):
    return pl.pallas_call(
        _t3_body,
        grid=(NB,),
        in_specs=[
            pl.BlockSpec((NC, BLK, D), lambda i: (0, i, 0)),
            pl.BlockSpec((BLK, D), lambda i: (i, 0)),
            pl.BlockSpec((BLK, 1), lambda i: (i, 0)),
            pl.BlockSpec((1, D), lambda i: (0, 0)),
            pl.BlockSpec((BLK, 1), lambda i: (i, 0)),
            pl.BlockSpec((3 * D, D), lambda i: (0, 0)),
            pl.BlockSpec((1, D), lambda i: (0, 0)),
        ],
        out_specs=pl.BlockSpec((G, D), lambda i: (0, 0)),
        out_shape=jax.ShapeDtypeStruct((G, D), jnp.float32),
        scratch_shapes=[
            pltpu.VMEM((G, D), jnp.float32),
            pltpu.VMEM((G, D), jnp.float32),
            pltpu.VMEM((G, D), jnp.float32),
        ],
    )(accp, hws, dis, b, br, Wp, bop)


@jax.jit
def kernel(x, edge_index, batch, W0, b0, W1, b1, W2, b2, Wout, bout):
    src = edge_index[0]
    dst = edge_index[1]
    zrows = jnp.zeros((RPT_LAST, D), jnp.float32)
    ones_c = jnp.ones((CHUNK, DEGW), jnp.float32)
    znodes = jnp.zeros((N, DEGW), jnp.float32)

    degp = _sc_degree(dst, ones_c, znodes)
    degT = degp[:, :, 0].T

    dis, hws = _tc_first(degT, x, W0)

    b0r = b0.reshape(1, D)
    b1r = b1.reshape(1, D)
    b2r = b2.reshape(1, D)
    br = batch.reshape(N, 1)
    Wp = jnp.pad(Wout, ((0, 0), (0, D - OUT)))
    bop = jnp.pad(bout, (0, D - OUT)).reshape(1, D)

    accp = _sc_message(hws, src, dst, zrows)
    hws1 = _tc_layer(accp, hws, dis, b0r, W1)
    accp = _sc_message(hws1, src, dst, zrows)
    hws2 = _tc_layer(accp, hws1, dis, b1r, W2)
    accp = _sc_message(hws2, src, dst, zrows)
    out128 = _tc_pool_head(accp, hws2, dis, b2r, br, Wp, bop)
    return out128[:, :OUT]
